# baseline (speedup 1.0000x reference)
import functools

import jax
import jax.numpy as jnp
from jax import lax
from jax.experimental import pallas as pl
from jax.experimental.pallas import tpu as pltpu
from jax.experimental.pallas import tpu_sc as plsc


def _make_gather(NF, NW, NC, CR):
    r_per_w = NF // NW
    n_chunks = r_per_w // CR
    mesh = plsc.VectorSubcoreMesh(core_axis_name="c", subcore_axis_name="s")

    @functools.partial(
        pl.kernel,
        mesh=mesh,
        compiler_params=pltpu.CompilerParams(use_tc_tiling_on_sc=False),
        out_type=jax.ShapeDtypeStruct((NF, 128), jnp.float32),
        scratch_types=[
            pltpu.VMEM((r_per_w,), jnp.int32),
            pltpu.VMEM((CR, 128), jnp.float32),
            pltpu.SemaphoreType.DMA,
        ],
    )
    def k(idx_hbm, t128_hbm, out_hbm, idx_v, rows_v, sem):
        wid = lax.axis_index("s") * NC + lax.axis_index("c")
        base = wid * r_per_w
        pltpu.sync_copy(idx_hbm.at[pl.ds(wid * r_per_w, r_per_w)], idx_v)

        def body(c, carry):
            pltpu.async_copy(
                t128_hbm.at[idx_v.at[pl.ds(c * CR, CR)]], rows_v, sem
            ).wait()
            pltpu.sync_copy(rows_v, out_hbm.at[pl.ds(base + c * CR, CR)])
            return carry

        lax.fori_loop(0, n_chunks, body, 0)

    return k


def kernel(input_batch, table):
    B, H = input_batch.shape
    V, D = table.shape
    HP = 56

    info = plsc.get_sparse_core_info()
    NC, NS = info.num_cores, info.num_subcores
    NW = NC * NS
    CR = 512

    t128 = jnp.zeros((V, 2 * D), jnp.float32).at[:, :D].set(table)
    junk = (
        jnp.arange(B, dtype=jnp.int32)[:, None] * (HP - H)
        + jnp.arange(HP - H, dtype=jnp.int32)[None, :]
    )
    idxp = jnp.concatenate([input_batch.astype(jnp.int32), junk % V], axis=1)
    idxf = idxp.reshape(B * HP)
    out = _make_gather(B * HP, NW, NC, CR)(idxf, t128)
    return out.reshape(B, HP, 2 * D)[:, :H, :D]


# speedup vs baseline: 1.4789x; 1.4789x over previous
import functools

import jax
import jax.numpy as jnp
from jax import lax
from jax.experimental import pallas as pl
from jax.experimental.pallas import tpu as pltpu
from jax.experimental.pallas import tpu_sc as plsc


def _make_gather(NF, NW, NC, CR):
    r_per_w = NF // NW
    n_chunks = r_per_w // CR
    T = n_chunks // 2
    mesh = plsc.VectorSubcoreMesh(core_axis_name="c", subcore_axis_name="s")

    @functools.partial(
        pl.kernel,
        mesh=mesh,
        compiler_params=pltpu.CompilerParams(use_tc_tiling_on_sc=False),
        out_type=jax.ShapeDtypeStruct((NF, 128), jnp.float32),
        scratch_types=[
            pltpu.VMEM((r_per_w,), jnp.int32),
            pltpu.VMEM((CR, 128), jnp.float32),
            pltpu.VMEM((CR, 128), jnp.float32),
            pltpu.SemaphoreType.DMA,
            pltpu.SemaphoreType.DMA,
            pltpu.SemaphoreType.DMA,
            pltpu.SemaphoreType.DMA,
        ],
    )
    def k(idx_hbm, t128_hbm, out_hbm, idx_v, rows0, rows1, sg0, sg1, sw0, sw1):
        wid = lax.axis_index("s") * NC + lax.axis_index("c")
        base = wid * r_per_w
        pltpu.sync_copy(idx_hbm.at[pl.ds(wid * r_per_w, r_per_w)], idx_v)

        def gather(c, rows, sem):
            pltpu.async_copy(
                t128_hbm.at[idx_v.at[pl.ds(c * CR, CR)]], rows, sem
            )

        def gather_wait(rows, sem):
            pltpu.make_async_copy(
                t128_hbm.at[idx_v.at[pl.ds(0, CR)]], rows, sem
            ).wait()

        def write(c, rows, sem):
            pltpu.async_copy(rows, out_hbm.at[pl.ds(base + c * CR, CR)], sem)

        def write_wait(rows, sem):
            pltpu.make_async_copy(rows, out_hbm.at[pl.ds(base, CR)], sem).wait()

        gather(0, rows0, sg0)

        def body(t, carry):
            @pl.when(t > 0)
            def _():
                write_wait(rows1, sw1)

            gather(2 * t + 1, rows1, sg1)
            gather_wait(rows0, sg0)
            write(2 * t, rows0, sw0)
            gather_wait(rows1, sg1)

            @pl.when(t < T - 1)
            def _():
                write_wait(rows0, sw0)
                gather(2 * t + 2, rows0, sg0)

            write(2 * t + 1, rows1, sw1)
            return carry

        lax.fori_loop(0, T, body, 0)
        write_wait(rows0, sw0)
        write_wait(rows1, sw1)

    return k


def kernel(input_batch, table):
    B, H = input_batch.shape
    V, D = table.shape
    HP = 56

    info = plsc.get_sparse_core_info()
    NC, NS = info.num_cores, info.num_subcores
    NW = NC * NS
    CR = 448

    t128 = jnp.pad(table, ((0, 0), (0, D)))
    junk = (
        jnp.arange(B, dtype=jnp.int32)[:, None] * (HP - H)
        + jnp.arange(HP - H, dtype=jnp.int32)[None, :]
    )
    idxp = jnp.concatenate([input_batch.astype(jnp.int32), junk % V], axis=1)
    idxf = idxp.reshape(B * HP)
    out = _make_gather(B * HP, NW, NC, CR)(idxf, t128)
    return out.reshape(B, HP, 2 * D)[:, :H, :D]


# speedup vs baseline: 1.7903x; 1.2105x over previous
import functools

import jax
import jax.numpy as jnp
from jax import lax
from jax.experimental import pallas as pl
from jax.experimental.pallas import tpu as pltpu
from jax.experimental.pallas import tpu_sc as plsc


def _make_gather(NF, NW, NC, CR):
    r_per_w = NF // NW
    n_chunks = r_per_w // CR
    T = n_chunks // 2
    mesh = plsc.VectorSubcoreMesh(core_axis_name="c", subcore_axis_name="s")

    @functools.partial(
        pl.kernel,
        mesh=mesh,
        compiler_params=pltpu.CompilerParams(use_tc_tiling_on_sc=False),
        out_type=jax.ShapeDtypeStruct((NF, 128), jnp.float32),
        scratch_types=[
            pltpu.VMEM((r_per_w,), jnp.int32),
            pltpu.VMEM((CR, 128), jnp.float32),
            pltpu.VMEM((CR, 128), jnp.float32),
            pltpu.SemaphoreType.DMA,
            pltpu.SemaphoreType.DMA,
            pltpu.SemaphoreType.DMA,
            pltpu.SemaphoreType.DMA,
        ],
    )
    def k(idx_hbm, t128_hbm, out_hbm, idx_v, rows0, rows1, sg0, sg1, sw0, sw1):
        wid = lax.axis_index("s") * NC + lax.axis_index("c")
        base = wid * r_per_w
        pltpu.sync_copy(idx_hbm.at[pl.ds(wid * r_per_w, r_per_w)], idx_v)

        def gather(c, rows, sem):
            pltpu.async_copy(
                t128_hbm.at[idx_v.at[pl.ds(c * CR, CR)]], rows, sem
            )

        def gather_wait(rows, sem):
            pltpu.make_async_copy(
                t128_hbm.at[idx_v.at[pl.ds(0, CR)]], rows, sem
            ).wait()

        def write(c, rows, sem):
            pltpu.async_copy(rows, out_hbm.at[pl.ds(base + c * CR, CR)], sem)

        def write_wait(rows, sem):
            pltpu.make_async_copy(rows, out_hbm.at[pl.ds(base, CR)], sem).wait()

        gather(0, rows0, sg0)

        def body(t, carry):
            @pl.when(t > 0)
            def _():
                write_wait(rows1, sw1)

            gather(2 * t + 1, rows1, sg1)
            gather_wait(rows0, sg0)
            write(2 * t, rows0, sw0)
            gather_wait(rows1, sg1)

            @pl.when(t < T - 1)
            def _():
                write_wait(rows0, sw0)
                gather(2 * t + 2, rows0, sg0)

            write(2 * t + 1, rows1, sw1)
            return carry

        lax.fori_loop(0, T, body, 0)
        write_wait(rows0, sw0)
        write_wait(rows1, sw1)

    return k


def kernel(input_batch, table):
    B, H = input_batch.shape
    V, D = table.shape
    HP = 56

    info = plsc.get_sparse_core_info()
    NC, NS = info.num_cores, info.num_subcores
    NW = NC * NS
    CR = 448

    proj = jnp.eye(D, 2 * D, dtype=jnp.float32)
    t128 = jax.lax.dot(table, proj, precision=jax.lax.Precision.HIGHEST)
    junk = (
        jnp.arange(B, dtype=jnp.int32)[:, None] * (HP - H)
        + jnp.arange(HP - H, dtype=jnp.int32)[None, :]
    )
    idxp = jnp.concatenate([input_batch.astype(jnp.int32), junk % V], axis=1)
    idxf = idxp.reshape(B * HP)
    out = _make_gather(B * HP, NW, NC, CR)(idxf, t128)
    return out.reshape(B, HP, 2 * D)[:, :H, :D]
